# baseline (speedup 1.0000x reference)
import functools

import jax
import jax.numpy as jnp
from jax import lax
from jax.experimental import pallas as pl
from jax.experimental.pallas import tpu as pltpu
from jax.experimental.pallas import tpu_sc as plsc

_NC = 2
_NS = 16
_NW = _NC * _NS

_IDX_CHUNK = 128


def _sc_gather(tok_table, idx3d, numchar, batch, emb_dim):
    n_rows = batch * numchar
    chunks = batch // _IDX_CHUNK
    mesh = plsc.VectorSubcoreMesh(core_axis_name="c", subcore_axis_name="s")

    @functools.partial(
        pl.kernel,
        mesh=mesh,
        out_type=jax.ShapeDtypeStruct((n_rows, emb_dim), jnp.float32),
        scratch_types=[
            pltpu.VMEM((chunks, _IDX_CHUNK), jnp.int32),
            pltpu.VMEM((chunks, _IDX_CHUNK), jnp.int32),
            pltpu.VMEM((batch, emb_dim), jnp.float32),
            pltpu.SemaphoreType.DMA,
            pltpu.SemaphoreType.DMA,
        ],
        compiler_params=pltpu.CompilerParams(use_tc_tiling_on_sc=False),
    )
    def gather_kernel(table_hbm, idx_hbm, out_hbm, idx_v, dst_v, rows_v, g_sem, s_sem):
        wid = lax.axis_index("s") * _NC + lax.axis_index("c")

        @pl.when(wid < numchar)
        def _():
            pltpu.sync_copy(idx_hbm.at[wid], idx_v)
            gathers = []
            for j in range(chunks):
                gathers.append(
                    pltpu.async_copy(
                        table_hbm.at[idx_v.at[j]],
                        rows_v.at[pl.ds(j * _IDX_CHUNK, _IDX_CHUNK)],
                        g_sem,
                    )
                )
            lane = lax.iota(jnp.int32, 16)
            for j in range(chunks):
                for k in range(_IDX_CHUNK // 16):
                    base = (j * _IDX_CHUNK + k * 16) * numchar
                    dst_v[j, pl.ds(k * 16, 16)] = lane * numchar + (base + wid)
            for g in gathers:
                g.wait()
            scatters = []
            for j in range(chunks):
                scatters.append(
                    pltpu.async_copy(
                        rows_v.at[pl.ds(j * _IDX_CHUNK, _IDX_CHUNK)],
                        out_hbm.at[dst_v.at[j]],
                        s_sem,
                    )
                )
            for s in scatters:
                s.wait()

    return gather_kernel(tok_table, idx3d)


def _tc_head_body(x_ref, pos_ref, wt_ref, b_ref, o_ref, xb_ref):
    @pl.when(pl.program_id(0) == 0)
    def _():
        xb_ref[...] = (x_ref[...] + pos_ref[...]).astype(jnp.bfloat16)

    w = wt_ref[...].astype(jnp.bfloat16)
    acc = lax.dot_general(
        w, xb_ref[...], (((0,), (1,)), ((), ())),
        preferred_element_type=jnp.float32,
    )
    o_ref[...] = acc + b_ref[...][:, None]


def _tc_head(x, pos_flat, WT, b, vt=2048):
    batch, k = x.shape
    vocab = WT.shape[1]
    grid = (vocab + vt - 1) // vt
    return pl.pallas_call(
        _tc_head_body,
        grid=(grid,),
        in_specs=[
            pl.BlockSpec((batch, k), lambda i: (0, 0)),
            pl.BlockSpec((1, k), lambda i: (0, 0)),
            pl.BlockSpec((k, vt), lambda i: (0, i)),
            pl.BlockSpec((vt,), lambda i: (i,)),
        ],
        out_specs=pl.BlockSpec((vt, batch), lambda i: (i, 0)),
        out_shape=jax.ShapeDtypeStruct((vocab, batch), jnp.float32),
        scratch_shapes=[pltpu.VMEM((batch, k), jnp.bfloat16)],
        compiler_params=pltpu.CompilerParams(
            dimension_semantics=("arbitrary",),
        ),
    )(x, pos_flat, WT, b)


def kernel(input_tokens, tok_table, pos_table, W, b):
    batch, numchar = input_tokens.shape
    emb_dim = tok_table.shape[1]
    idx3d = input_tokens.T.reshape(numchar, batch // _IDX_CHUNK, _IDX_CHUNK)
    emb = _sc_gather(tok_table, idx3d, numchar, batch, emb_dim)
    x = emb.reshape(batch, numchar * emb_dim)
    pos_flat = pos_table.reshape(1, numchar * emb_dim)
    out_t = _tc_head(x, pos_flat, W.T, b)
    return out_t.T


# speedup vs baseline: 1.0220x; 1.0220x over previous
import functools

import jax
import jax.numpy as jnp
from jax import lax
from jax.experimental import pallas as pl
from jax.experimental.pallas import tpu as pltpu
from jax.experimental.pallas import tpu_sc as plsc

_NC = 2
_NS = 16
_NW = _NC * _NS

_IDX_CHUNK = 128


def _sc_gather(tok_table, idx3d, numchar, batch, emb_dim):
    n_rows = batch * numchar
    chunks = batch // _IDX_CHUNK
    mesh = plsc.VectorSubcoreMesh(core_axis_name="c", subcore_axis_name="s")

    @functools.partial(
        pl.kernel,
        mesh=mesh,
        out_type=jax.ShapeDtypeStruct((n_rows, emb_dim), jnp.float32),
        scratch_types=[
            pltpu.VMEM((chunks, _IDX_CHUNK), jnp.int32),
            pltpu.VMEM((chunks, _IDX_CHUNK), jnp.int32),
            pltpu.VMEM((batch, emb_dim), jnp.float32),
            pltpu.SemaphoreType.DMA,
            pltpu.SemaphoreType.DMA,
        ],
        compiler_params=pltpu.CompilerParams(use_tc_tiling_on_sc=False),
    )
    def gather_kernel(table_hbm, idx_hbm, out_hbm, idx_v, dst_v, rows_v, g_sem, s_sem):
        wid = lax.axis_index("s") * _NC + lax.axis_index("c")

        @pl.when(wid < numchar)
        def _():
            pltpu.sync_copy(idx_hbm.at[wid], idx_v)
            gathers = []
            for j in range(chunks):
                gathers.append(
                    pltpu.async_copy(
                        table_hbm.at[idx_v.at[j]],
                        rows_v.at[pl.ds(j * _IDX_CHUNK, _IDX_CHUNK)],
                        g_sem,
                    )
                )
            lane = lax.iota(jnp.int32, 16)
            for j in range(chunks):
                for k in range(_IDX_CHUNK // 16):
                    base = (j * _IDX_CHUNK + k * 16) * numchar
                    dst_v[j, pl.ds(k * 16, 16)] = lane * numchar + (base + wid)
            for g in gathers:
                g.wait()
            scatters = []
            for j in range(chunks):
                scatters.append(
                    pltpu.async_copy(
                        rows_v.at[pl.ds(j * _IDX_CHUNK, _IDX_CHUNK)],
                        out_hbm.at[dst_v.at[j]],
                        s_sem,
                    )
                )
            for s in scatters:
                s.wait()

    return gather_kernel(tok_table, idx3d)


def _tc_head_body(x_ref, pos_ref, wt_ref, b_ref, o_ref, xb_ref):
    @pl.when(pl.program_id(0) == 0)
    def _():
        xb_ref[...] = (x_ref[...] + pos_ref[...]).astype(jnp.bfloat16)

    w = wt_ref[...].astype(jnp.bfloat16)
    acc = lax.dot_general(
        w, xb_ref[...], (((0,), (1,)), ((), ())),
        preferred_element_type=jnp.float32,
    )
    o_ref[...] = acc + b_ref[...][:, None]


def _tc_head(x, pos_flat, WT, b, vt=4096):
    batch, k = x.shape
    vocab = WT.shape[1]
    grid = (vocab + vt - 1) // vt
    return pl.pallas_call(
        _tc_head_body,
        grid=(grid,),
        in_specs=[
            pl.BlockSpec((batch, k), lambda i: (0, 0)),
            pl.BlockSpec((1, k), lambda i: (0, 0)),
            pl.BlockSpec((k, vt), lambda i: (0, i)),
            pl.BlockSpec((vt,), lambda i: (i,)),
        ],
        out_specs=pl.BlockSpec((vt, batch), lambda i: (i, 0)),
        out_shape=jax.ShapeDtypeStruct((vocab, batch), jnp.float32),
        scratch_shapes=[pltpu.VMEM((batch, k), jnp.bfloat16)],
        compiler_params=pltpu.CompilerParams(
            dimension_semantics=("parallel",),
        ),
    )(x, pos_flat, WT, b)


def kernel(input_tokens, tok_table, pos_table, W, b):
    batch, numchar = input_tokens.shape
    emb_dim = tok_table.shape[1]
    idx3d = input_tokens.T.reshape(numchar, batch // _IDX_CHUNK, _IDX_CHUNK)
    emb = _sc_gather(tok_table, idx3d, numchar, batch, emb_dim)
    x = emb.reshape(batch, numchar * emb_dim)
    pos_flat = pos_table.reshape(1, numchar * emb_dim)
    out_t = _tc_head(x, pos_flat, W.T, b)
    return out_t.T


# speedup vs baseline: 1.0508x; 1.0282x over previous
import functools

import jax
import jax.numpy as jnp
from jax import lax
from jax.experimental import pallas as pl
from jax.experimental.pallas import tpu as pltpu
from jax.experimental.pallas import tpu_sc as plsc

_NC = 2
_NS = 16
_NW = _NC * _NS

_IDX_CHUNK = 128


def _sc_gather_t(table_t, idx3d, pos_table, numchar, batch, emb_dim):
    chunks = batch // _IDX_CHUNK
    mesh = plsc.VectorSubcoreMesh(core_axis_name="c", subcore_axis_name="s")

    @functools.partial(
        pl.kernel,
        mesh=mesh,
        out_type=jax.ShapeDtypeStruct((numchar * emb_dim, batch), jnp.float32),
        scratch_types=[
            pltpu.VMEM((chunks, _IDX_CHUNK), jnp.int32),
            pltpu.VMEM((emb_dim, batch), jnp.float32),
            pltpu.VMEM((emb_dim,), jnp.float32),
            pltpu.SemaphoreType.DMA,
        ],
        compiler_params=pltpu.CompilerParams(use_tc_tiling_on_sc=False),
    )
    def gather_kernel(table_hbm, idx_hbm, pos_hbm, out_hbm, idx_v, xt_v, pos_v, sem):
        wid = lax.axis_index("s") * _NC + lax.axis_index("c")

        @pl.when(wid < numchar)
        def _():
            pltpu.sync_copy(idx_hbm.at[wid], idx_v)
            pltpu.sync_copy(pos_hbm.at[wid], pos_v)
            for j in range(chunks):
                copies = []
                for d in range(emb_dim):
                    copies.append(
                        pltpu.async_copy(
                            table_hbm.at[d].at[idx_v.at[j]],
                            xt_v.at[d, pl.ds(j * _IDX_CHUNK, _IDX_CHUNK)],
                            sem,
                        )
                    )
                for c in copies:
                    c.wait()
            pv_vec = pos_v[...]
            for d in range(emb_dim):
                pv = pv_vec[d]

                def add_chunk(k, _, d=d, pv=pv):
                    sl = pl.ds(k * 16, 16)
                    xt_v[d, sl] = xt_v[d, sl] + pv
                    return _

                lax.fori_loop(0, batch // 16, add_chunk, None)
            pltpu.sync_copy(xt_v, out_hbm.at[pl.ds(wid * emb_dim, emb_dim)])

    return gather_kernel(table_t, idx3d, pos_table)


def _tc_head_body(xt_ref, wt_ref, b_ref, o_ref, xb_ref):
    @pl.when(pl.program_id(0) == 0)
    def _():
        xb_ref[...] = xt_ref[...].astype(jnp.bfloat16)

    w = wt_ref[...].astype(jnp.bfloat16)
    acc = lax.dot_general(
        w, xb_ref[...], (((0,), (0,)), ((), ())),
        preferred_element_type=jnp.float32,
    )
    o_ref[...] = acc + b_ref[...][:, None]


def _tc_head(x_t, WT, b, vt=4096):
    k, batch = x_t.shape
    vocab = WT.shape[1]
    grid = (vocab + vt - 1) // vt
    return pl.pallas_call(
        _tc_head_body,
        grid=(grid,),
        in_specs=[
            pl.BlockSpec((k, batch), lambda i: (0, 0)),
            pl.BlockSpec((k, vt), lambda i: (0, i)),
            pl.BlockSpec((vt,), lambda i: (i,)),
        ],
        out_specs=pl.BlockSpec((vt, batch), lambda i: (i, 0)),
        out_shape=jax.ShapeDtypeStruct((vocab, batch), jnp.float32),
        scratch_shapes=[pltpu.VMEM((k, batch), jnp.bfloat16)],
        compiler_params=pltpu.CompilerParams(
            dimension_semantics=("parallel",),
        ),
    )(x_t, WT, b)


def kernel(input_tokens, tok_table, pos_table, W, b):
    batch, numchar = input_tokens.shape
    emb_dim = tok_table.shape[1]
    idx3d = input_tokens.T.reshape(numchar, batch // _IDX_CHUNK, _IDX_CHUNK)
    x_t = _sc_gather_t(tok_table.T, idx3d, pos_table, numchar, batch, emb_dim)
    out_t = _tc_head(x_t, W.T, b)
    return out_t.T


# speedup vs baseline: 1.0727x; 1.0208x over previous
import functools

import jax
import jax.numpy as jnp
from jax import lax
from jax.experimental import pallas as pl
from jax.experimental.pallas import tpu as pltpu
from jax.experimental.pallas import tpu_sc as plsc

_NC = 2
_NS = 16
_NW = _NC * _NS

_IDX_CHUNK = 128


def _sc_gather_t(table_t, idx3d, pos_table, numchar, batch, emb_dim):
    chunks = batch // _IDX_CHUNK
    mesh = plsc.VectorSubcoreMesh(core_axis_name="c", subcore_axis_name="s")

    @functools.partial(
        pl.kernel,
        mesh=mesh,
        out_type=jax.ShapeDtypeStruct((numchar * emb_dim, batch), jnp.float32),
        scratch_types=[
            pltpu.VMEM((chunks, _IDX_CHUNK), jnp.int32),
            pltpu.VMEM((emb_dim, batch), jnp.float32),
            pltpu.VMEM((emb_dim,), jnp.float32),
            pltpu.SemaphoreType.DMA,
        ],
        compiler_params=pltpu.CompilerParams(use_tc_tiling_on_sc=False),
    )
    def gather_kernel(table_hbm, idx_hbm, pos_hbm, out_hbm, idx_v, xt_v, pos_v, sem):
        wid = lax.axis_index("s") * _NC + lax.axis_index("c")

        @pl.when(wid < numchar)
        def _():
            pltpu.sync_copy(idx_hbm.at[wid], idx_v)
            pltpu.sync_copy(pos_hbm.at[wid], pos_v)
            copies = []
            for j in range(chunks):
                for d in range(emb_dim):
                    copies.append(
                        pltpu.async_copy(
                            table_hbm.at[d].at[idx_v.at[j]],
                            xt_v.at[d, pl.ds(j * _IDX_CHUNK, _IDX_CHUNK)],
                            sem,
                        )
                    )
            for c in copies:
                c.wait()
            pv_vec = pos_v[...]
            for d in range(emb_dim):
                pv = pv_vec[d]

                def add_chunk(k, _, d=d, pv=pv):
                    sl = pl.ds(k * 16, 16)
                    xt_v[d, sl] = xt_v[d, sl] + pv
                    return _

                lax.fori_loop(0, batch // 16, add_chunk, None)
            pltpu.sync_copy(xt_v, out_hbm.at[pl.ds(wid * emb_dim, emb_dim)])

    return gather_kernel(table_t, idx3d, pos_table)


def _tc_head_body(xt_ref, wt_ref, b_ref, o_ref, xb_ref):
    @pl.when(pl.program_id(0) == 0)
    def _():
        xb_ref[...] = xt_ref[...].astype(jnp.bfloat16)

    w = wt_ref[...].astype(jnp.bfloat16)
    acc = lax.dot_general(
        w, xb_ref[...], (((0,), (0,)), ((), ())),
        preferred_element_type=jnp.float32,
    )
    o_ref[...] = acc + b_ref[...][:, None]


def _tc_head(x_t, WT, b, vt=4096):
    k, batch = x_t.shape
    vocab = WT.shape[1]
    grid = (vocab + vt - 1) // vt
    return pl.pallas_call(
        _tc_head_body,
        grid=(grid,),
        in_specs=[
            pl.BlockSpec((k, batch), lambda i: (0, 0)),
            pl.BlockSpec((k, vt), lambda i: (0, i)),
            pl.BlockSpec((vt,), lambda i: (i,)),
        ],
        out_specs=pl.BlockSpec((vt, batch), lambda i: (i, 0)),
        out_shape=jax.ShapeDtypeStruct((vocab, batch), jnp.float32),
        scratch_shapes=[pltpu.VMEM((k, batch), jnp.bfloat16)],
        compiler_params=pltpu.CompilerParams(
            dimension_semantics=("parallel",),
        ),
    )(x_t, WT, b)


def kernel(input_tokens, tok_table, pos_table, W, b):
    batch, numchar = input_tokens.shape
    emb_dim = tok_table.shape[1]
    idx3d = input_tokens.T.reshape(numchar, batch // _IDX_CHUNK, _IDX_CHUNK)
    x_t = _sc_gather_t(tok_table.T, idx3d, pos_table, numchar, batch, emb_dim)
    out_t = _tc_head(x_t, W.T, b)
    return out_t.T


# speedup vs baseline: 1.1378x; 1.0607x over previous
import functools

import jax
import jax.numpy as jnp
from jax import lax
from jax.experimental import pallas as pl
from jax.experimental.pallas import tpu as pltpu
from jax.experimental.pallas import tpu_sc as plsc

_NC = 2
_NS = 16
_NW = _NC * _NS

_IDX_CHUNK = 128


def _sc_gather_t(table_t, idx3d, numchar, batch, emb_dim):
    chunks = batch // _IDX_CHUNK
    rows_per_w = numchar * emb_dim // _NW
    c_step = _NW // emb_dim
    mesh = plsc.VectorSubcoreMesh(core_axis_name="c", subcore_axis_name="s")

    @functools.partial(
        pl.kernel,
        mesh=mesh,
        out_type=jax.ShapeDtypeStruct((numchar * emb_dim, batch), jnp.float32),
        scratch_types=[
            pltpu.VMEM((rows_per_w, chunks, _IDX_CHUNK), jnp.int32),
            pltpu.VMEM((rows_per_w, batch), jnp.float32),
            pltpu.SemaphoreType.DMA,
            pltpu.SemaphoreType.DMA,
        ],
        compiler_params=pltpu.CompilerParams(use_tc_tiling_on_sc=False),
    )
    def gather_kernel(table_hbm, idx_hbm, out_hbm, idx_v, xt_v, i_sem, sem):
        wid = lax.axis_index("s") * _NC + lax.axis_index("c")
        d = lax.rem(wid, emb_dim)
        c0 = wid // emb_dim
        stages = []
        for k in range(rows_per_w):
            stages.append(
                pltpu.async_copy(idx_hbm.at[c0 + c_step * k], idx_v.at[k], i_sem)
            )
        for s in stages:
            s.wait()
        copies = []
        for k in range(rows_per_w):
            for j in range(chunks):
                copies.append(
                    pltpu.async_copy(
                        table_hbm.at[d].at[idx_v.at[k].at[j]],
                        xt_v.at[k, pl.ds(j * _IDX_CHUNK, _IDX_CHUNK)],
                        sem,
                    )
                )
        for c in copies:
            c.wait()
        writes = []
        for k in range(rows_per_w):
            row = (c0 + c_step * k) * emb_dim + d
            writes.append(pltpu.async_copy(xt_v.at[k], out_hbm.at[row], i_sem))
        for wr in writes:
            wr.wait()

    return gather_kernel(table_t, idx3d)


def _tc_head_body(xt_ref, pos_ref, wt_ref, b_ref, o_ref, xb_ref):
    @pl.when(pl.program_id(0) == 0)
    def _():
        xb_ref[...] = (xt_ref[...] + pos_ref[...]).astype(jnp.bfloat16)

    w = wt_ref[...].astype(jnp.bfloat16)
    acc = lax.dot_general(
        w, xb_ref[...], (((0,), (0,)), ((), ())),
        preferred_element_type=jnp.float32,
    )
    o_ref[...] = acc + b_ref[...][:, None]


def _tc_head(x_t, pos_col, WT, b, vt=4096):
    k, batch = x_t.shape
    vocab = WT.shape[1]
    grid = (vocab + vt - 1) // vt
    return pl.pallas_call(
        _tc_head_body,
        grid=(grid,),
        in_specs=[
            pl.BlockSpec((k, batch), lambda i: (0, 0)),
            pl.BlockSpec((k, 1), lambda i: (0, 0)),
            pl.BlockSpec((k, vt), lambda i: (0, i)),
            pl.BlockSpec((vt,), lambda i: (i,)),
        ],
        out_specs=pl.BlockSpec((vt, batch), lambda i: (i, 0)),
        out_shape=jax.ShapeDtypeStruct((vocab, batch), jnp.float32),
        scratch_shapes=[pltpu.VMEM((k, batch), jnp.bfloat16)],
        compiler_params=pltpu.CompilerParams(
            dimension_semantics=("parallel",),
        ),
    )(x_t, pos_col, WT, b)


def kernel(input_tokens, tok_table, pos_table, W, b):
    batch, numchar = input_tokens.shape
    emb_dim = tok_table.shape[1]
    idx3d = input_tokens.T.reshape(numchar, batch // _IDX_CHUNK, _IDX_CHUNK)
    x_t = _sc_gather_t(tok_table.T, idx3d, numchar, batch, emb_dim)
    pos_col = pos_table.reshape(numchar * emb_dim, 1)
    out_t = _tc_head(x_t, pos_col, W.T, b)
    return out_t.T
